# baseline (speedup 1.0000x reference)
import jax
import jax.numpy as jnp
from jax import lax
from jax.experimental import pallas as pl
from jax.experimental.pallas import tpu as pltpu
from jax.experimental.pallas import tpu_sc as plsc

_WINDOW = 128
_R = 2048
_TC_FRAC = 0.0


def _tc_lookup(idx_tc, table, n_out):
    m = idx_tc.shape[0]
    vocab, dim = table.shape
    nblk = m // _R
    idx3 = idx_tc.reshape(nblk, 1, _R)

    tpad = jnp.zeros((128, dim), table.dtype).at[:vocab].set(table)
    thi = tpad.astype(jnp.bfloat16)
    tlo = (tpad - thi.astype(jnp.float32)).astype(jnp.bfloat16)

    def body(idx_ref, thi_ref, tlo_ref, o_ref):
        ids = idx_ref[0, 0, :]
        oh = (ids[:, None] == lax.broadcasted_iota(jnp.int32, (_R, 128), 1)).astype(
            jnp.bfloat16
        )
        o_ref[...] = jnp.dot(
            oh, thi_ref[...], preferred_element_type=jnp.float32
        ) + jnp.dot(oh, tlo_ref[...], preferred_element_type=jnp.float32)

    return pl.pallas_call(
        body,
        grid=(nblk,),
        in_specs=[
            pl.BlockSpec((1, 1, _R), lambda i: (i, 0, 0)),
            pl.BlockSpec((128, dim), lambda i: (0, 0)),
            pl.BlockSpec((128, dim), lambda i: (0, 0)),
        ],
        out_specs=pl.BlockSpec((_R, dim), lambda i: (i, 0)),
        out_shape=jax.ShapeDtypeStruct((n_out, dim), table.dtype),
    )(idx3, thi, tlo)


def _sc_lookup(idx_sc, table):
    m = idx_sc.shape[0]
    vocab, dim = table.shape
    np_ = m // 2

    pid = (idx_sc.reshape(np_, 2)[:, 0] * vocab + idx_sc.reshape(np_, 2)[:, 1]).reshape(
        1, np_
    )
    table2 = jnp.concatenate(
        [
            jnp.broadcast_to(table[:, None, :], (vocab, vocab, dim)),
            jnp.broadcast_to(table[None, :, :], (vocab, vocab, dim)),
        ],
        axis=-1,
    ).reshape(vocab * vocab, 2 * dim)

    mesh = plsc.VectorSubcoreMesh(core_axis_name="c", subcore_axis_name="s")

    @pl.kernel(out_type=jax.ShapeDtypeStruct((np_, 2 * dim), table.dtype), mesh=mesh)
    def _gather(table_hbm, idx_hbm, out_hbm):
        def body(i_vmem, o_vmem):
            pltpu.sync_copy(table_hbm.at[i_vmem.at[0]], o_vmem)

        pltpu.emit_pipeline(
            body,
            grid=(np_ // _WINDOW,),
            in_specs=[pl.BlockSpec((1, _WINDOW), index_map=lambda i: (0, i))],
            out_specs=[pl.BlockSpec((_WINDOW, 2 * dim), index_map=lambda i: (i, 0))],
            core_axis_name=("c", "s"),
            dimension_semantics=(pltpu.PARALLEL,),
        )(idx_hbm, out_hbm)

    return _gather(table2, pid).reshape(m, dim)


def kernel(poi_categories, table):
    batch, seq = poi_categories.shape
    vocab, dim = table.shape
    n = batch * seq
    idx = poi_categories.reshape(n).astype(jnp.int32)

    n_tc = int(n * _TC_FRAC) // 8192 * 8192
    n_sc = n - n_tc

    if n_sc == 0:
        out = _tc_lookup(idx, table, n)
    elif n_tc == 0:
        out = _sc_lookup(idx, table)
    else:
        out_tc = _tc_lookup(idx[:n_tc], table, n)
        out_sc = _sc_lookup(idx[n_tc:], table)
        out = lax.dynamic_update_slice(out_tc, out_sc, (n_tc, 0))
    return out.reshape(batch, seq, dim)


# speedup vs baseline: 2.9192x; 2.9192x over previous
import dataclasses

import jax
import jax.numpy as jnp
from jax import lax
from jax.experimental import pallas as pl
from jax.experimental.pallas import tpu as pltpu
from jax.experimental.pallas import tpu_sc as plsc

_W = 128
_PV = 88


def _build_pair_table(table):
    vocab, dim = table.shape
    tpad = jnp.pad(table, ((0, _PV - vocab), (0, 0)))

    def body(ta_ref, tb_ref, o_ref):
        o_ref[:, :dim] = jnp.broadcast_to(ta_ref[0], (_PV, dim))
        o_ref[:, dim:] = tb_ref[...]

    return pl.pallas_call(
        body,
        grid=(vocab,),
        in_specs=[
            pl.BlockSpec((1, 1, dim), lambda a: (a, 0, 0)),
            pl.BlockSpec((_PV, dim), lambda a: (0, 0)),
        ],
        out_specs=pl.BlockSpec((_PV, 2 * dim), lambda a: (a, 0)),
        out_shape=jax.ShapeDtypeStruct((vocab * _PV, 2 * dim), table.dtype),
    )(tpad.reshape(_PV, 1, dim), tpad)


def kernel(poi_categories, table):
    batch, seq = poi_categories.shape
    vocab, dim = table.shape
    n = batch * seq
    np_ = n // 2

    idx = poi_categories.reshape(1, n).astype(jnp.int32)
    table2 = _build_pair_table(table)

    mesh = plsc.VectorSubcoreMesh(core_axis_name="c", subcore_axis_name="s")
    cp = pltpu.CompilerParams()
    if "needs_layout_passes" in pltpu.CompilerParams.__dataclass_fields__:
        cp = dataclasses.replace(cp, needs_layout_passes=False)

    @pl.kernel(
        out_type=jax.ShapeDtypeStruct((n, dim), table.dtype),
        mesh=mesh,
        scratch_types=[pltpu.VMEM((_W,), jnp.int32)],
        compiler_params=cp,
    )
    def _gather(table_hbm, idx_hbm, out_hbm, pid_ref):
        iota = lax.iota(jnp.int32, 16)
        zeros = jnp.zeros((16,), jnp.int32)

        def body(i_vmem, o_vmem):
            for g in range(_W // 16):
                p = g * 16 + iota
                a = plsc.load_gather(i_vmem, [zeros, 2 * p])
                b = plsc.load_gather(i_vmem, [zeros, 2 * p + 1])
                pid_ref[pl.ds(g * 16, 16)] = a * _PV + b
            pltpu.sync_copy(table_hbm.at[pid_ref], o_vmem.reshape(_W, 2 * dim))

        pltpu.emit_pipeline(
            body,
            grid=(np_ // _W,),
            in_specs=[pl.BlockSpec((1, 2 * _W), index_map=lambda i: (0, i))],
            out_specs=[pl.BlockSpec((2 * _W, dim), index_map=lambda i: (i, 0))],
            core_axis_name=("c", "s"),
            dimension_semantics=(pltpu.PARALLEL,),
        )(idx_hbm, out_hbm)

    out = _gather(table2, idx)
    return out.reshape(batch, seq, dim)


# speedup vs baseline: 2.9409x; 1.0074x over previous
import dataclasses

import jax
import jax.numpy as jnp
from jax import lax
from jax.experimental import pallas as pl
from jax.experimental.pallas import tpu as pltpu
from jax.experimental.pallas import tpu_sc as plsc

_W = 128
_PV = 88


def _build_pair_table(table):
    vocab, dim = table.shape
    tpad = jnp.pad(table, ((0, _PV - vocab), (0, 0)))

    def body(ta_ref, tb_ref, o_ref):
        o_ref[:, :dim] = jnp.broadcast_to(ta_ref[0], (_PV, dim))
        o_ref[:, dim:] = tb_ref[...]

    return pl.pallas_call(
        body,
        grid=(vocab,),
        in_specs=[
            pl.BlockSpec((1, 1, dim), lambda a: (a, 0, 0)),
            pl.BlockSpec((_PV, dim), lambda a: (0, 0)),
        ],
        out_specs=pl.BlockSpec((_PV, 2 * dim), lambda a: (a, 0)),
        out_shape=jax.ShapeDtypeStruct((vocab * _PV, 2 * dim), table.dtype),
    )(tpad.reshape(_PV, 1, dim), tpad)


def kernel(poi_categories, table):
    batch, seq = poi_categories.shape
    vocab, dim = table.shape
    n = batch * seq
    np_ = n // 2

    idx = poi_categories.reshape(1, n).astype(jnp.int32)
    table2 = _build_pair_table(table)

    mesh = plsc.VectorSubcoreMesh(core_axis_name="c", subcore_axis_name="s")
    cp = pltpu.CompilerParams()
    if "needs_layout_passes" in pltpu.CompilerParams.__dataclass_fields__:
        cp = dataclasses.replace(cp, needs_layout_passes=False)

    nw = 32
    per_w = np_ // nw
    ic = 3200
    nsc = per_w // ic
    isteps = ic // _W

    @pl.kernel(
        out_type=jax.ShapeDtypeStruct((n, dim), table.dtype),
        mesh=mesh,
        scratch_types=[
            pltpu.VMEM((2 * 2 * ic,), jnp.int32),
            pltpu.VMEM((_W,), jnp.int32),
            pltpu.VMEM((_W,), jnp.int32),
            pltpu.VMEM((2 * _W, dim), jnp.float32),
            pltpu.VMEM((2 * _W, dim), jnp.float32),
            pltpu.SemaphoreType.DMA((2,)),
            pltpu.SemaphoreType.DMA((2,)),
            pltpu.SemaphoreType.DMA((2,)),
        ],
        compiler_params=cp,
    )
    def _gather(
        table_hbm, idx_hbm, out_hbm, idx_v, pid0_v, pid1_v, rows0_v, rows1_v,
        isem, gsem, wsem,
    ):
        pid_bufs = (pid0_v, pid1_v)
        rows_bufs = (rows0_v, rows1_v)
        wid = lax.axis_index("s") * 2 + lax.axis_index("c")
        base_i = wid * per_w * 2
        iota = lax.iota(jnp.int32, 16)

        def idx_load(c, jc):
            return pltpu.make_async_copy(
                idx_hbm.at[pl.ds(base_i + c * 2 * ic, 2 * ic)],
                idx_v.at[pl.ds(jc * 2 * ic, 2 * ic)],
                isem.at[jc],
            )

        def compute_pid(jc, s, slot):
            for g in range(_W // 16):
                pos = jc * 2 * ic + 2 * (s * _W + g * 16 + iota)
                a = plsc.load_gather(idx_v, [pos])
                b = plsc.load_gather(idx_v, [pos + 1])
                pid_bufs[slot][pl.ds(g * 16, 16)] = a * _PV + b

        def gather(slot):
            return pltpu.make_async_copy(
                table_hbm.at[pid_bufs[slot]],
                rows_bufs[slot].reshape(_W, 2 * dim),
                gsem.at[slot],
            )

        def writeback(c, s, slot):
            dst = out_hbm.at[pl.ds(base_i + c * 2 * ic + s * 2 * _W, 2 * _W)]
            return pltpu.make_async_copy(rows_bufs[slot], dst, wsem.at[slot])

        idx_load(0, 0).start()

        @pl.loop(0, nsc)
        def _(c):
            jc = lax.rem(c, 2)
            idx_load(c, jc).wait()

            @pl.when(c + 1 < nsc)
            def _():
                idx_load(c + 1, lax.rem(c + 1, 2)).start()

            @pl.when(c > 0)
            def _():
                writeback(c, 0, 0).wait()

            compute_pid(jc, 0, 0)
            gather(0).start()

            @pl.loop(0, isteps - 1, step=2)
            def _(r):
                for b in range(2):
                    s = r + b
                    gather(b).wait()
                    writeback(c, s, b).start()
                    tb = 1 - b

                    @pl.when((c > 0) | (s >= 1))
                    def _(tb=tb):
                        writeback(c, 0, tb).wait()

                    compute_pid(jc, s + 1, tb)
                    gather(tb).start()

            gather(0).wait()
            writeback(c, isteps - 1, 0).start()

        writeback(0, 0, 0).wait()
        writeback(0, 0, 1).wait()

    out = _gather(table2, idx.reshape(n))
    return out.reshape(batch, seq, dim)


# speedup vs baseline: 3.0016x; 1.0206x over previous
import dataclasses

import jax
import jax.numpy as jnp
from jax import lax
from jax.experimental import pallas as pl
from jax.experimental.pallas import tpu as pltpu
from jax.experimental.pallas import tpu_sc as plsc

_W = 128
_PV = 88


def _build_pair_table(table):
    vocab, dim = table.shape

    def body(t_ref, o_ref):
        t = t_ref[...]
        tpad = jnp.concatenate([t, jnp.zeros((_PV - vocab, dim), t.dtype)], 0)
        o_ref[:, :dim] = jnp.broadcast_to(t[:, None, :], (vocab, _PV, dim)).reshape(
            vocab * _PV, dim
        )
        o_ref[:, dim:] = jnp.broadcast_to(tpad[None], (vocab, _PV, dim)).reshape(
            vocab * _PV, dim
        )

    return pl.pallas_call(
        body,
        out_shape=jax.ShapeDtypeStruct((vocab * _PV, 2 * dim), table.dtype),
    )(table)


def kernel(poi_categories, table):
    batch, seq = poi_categories.shape
    vocab, dim = table.shape
    n = batch * seq
    np_ = n // 2

    idx = poi_categories.reshape(1, n).astype(jnp.int32)
    table2 = _build_pair_table(table)

    mesh = plsc.VectorSubcoreMesh(core_axis_name="c", subcore_axis_name="s")
    cp = pltpu.CompilerParams()
    if "needs_layout_passes" in pltpu.CompilerParams.__dataclass_fields__:
        cp = dataclasses.replace(cp, needs_layout_passes=False)

    nw = 32
    per_w = np_ // nw
    ic = 3200
    nsc = per_w // ic
    isteps = ic // _W

    @pl.kernel(
        out_type=jax.ShapeDtypeStruct((n, dim), table.dtype),
        mesh=mesh,
        scratch_types=[
            pltpu.VMEM((2 * 2 * ic,), jnp.int32),
            pltpu.VMEM((_W,), jnp.int32),
            pltpu.VMEM((_W,), jnp.int32),
            pltpu.VMEM((2 * _W, dim), jnp.float32),
            pltpu.VMEM((2 * _W, dim), jnp.float32),
            pltpu.SemaphoreType.DMA((2,)),
            pltpu.SemaphoreType.DMA((2,)),
            pltpu.SemaphoreType.DMA((2,)),
        ],
        compiler_params=cp,
    )
    def _gather(
        table_hbm, idx_hbm, out_hbm, idx_v, pid0_v, pid1_v, rows0_v, rows1_v,
        isem, gsem, wsem,
    ):
        pid_bufs = (pid0_v, pid1_v)
        rows_bufs = (rows0_v, rows1_v)
        wid = lax.axis_index("s") * 2 + lax.axis_index("c")
        base_i = wid * per_w * 2
        iota = lax.iota(jnp.int32, 16)

        def idx_load(c, jc):
            return pltpu.make_async_copy(
                idx_hbm.at[pl.ds(base_i + c * 2 * ic, 2 * ic)],
                idx_v.at[pl.ds(jc * 2 * ic, 2 * ic)],
                isem.at[jc],
            )

        def compute_pid(jc, s, slot):
            for g in range(_W // 16):
                pos = jc * 2 * ic + 2 * (s * _W + g * 16 + iota)
                a = plsc.load_gather(idx_v, [pos])
                b = plsc.load_gather(idx_v, [pos + 1])
                pid_bufs[slot][pl.ds(g * 16, 16)] = a * _PV + b

        def gather(slot):
            return pltpu.make_async_copy(
                table_hbm.at[pid_bufs[slot]],
                rows_bufs[slot].reshape(_W, 2 * dim),
                gsem.at[slot],
            )

        def writeback(c, s, slot):
            dst = out_hbm.at[pl.ds(base_i + c * 2 * ic + s * 2 * _W, 2 * _W)]
            return pltpu.make_async_copy(rows_bufs[slot], dst, wsem.at[slot])

        idx_load(0, 0).start()

        @pl.loop(0, nsc)
        def _(c):
            jc = lax.rem(c, 2)
            idx_load(c, jc).wait()

            @pl.when(c + 1 < nsc)
            def _():
                idx_load(c + 1, lax.rem(c + 1, 2)).start()

            @pl.when(c > 0)
            def _():
                writeback(c, 0, 0).wait()

            compute_pid(jc, 0, 0)
            gather(0).start()

            @pl.loop(0, isteps - 1, step=2)
            def _(r):
                for b in range(2):
                    s = r + b
                    gather(b).wait()
                    writeback(c, s, b).start()
                    tb = 1 - b

                    @pl.when((c > 0) | (s >= 1))
                    def _(tb=tb):
                        writeback(c, 0, tb).wait()

                    compute_pid(jc, s + 1, tb)
                    gather(tb).start()

            gather(0).wait()
            writeback(c, isteps - 1, 0).start()

        writeback(0, 0, 0).wait()
        writeback(0, 0, 1).wait()

    out = _gather(table2, idx.reshape(n))
    return out.reshape(batch, seq, dim)


# speedup vs baseline: 3.0106x; 1.0030x over previous
import dataclasses

import jax
import jax.numpy as jnp
from jax import lax
from jax.experimental import pallas as pl
from jax.experimental.pallas import tpu as pltpu
from jax.experimental.pallas import tpu_sc as plsc

_W = 128
_PV = 88


def _build_pair_table(table):
    vocab, dim = table.shape

    def body(t_ref, o_ref):
        t = t_ref[...]
        tpad = jnp.concatenate([t, jnp.zeros((_PV - vocab, dim), t.dtype)], 0)
        o_ref[:, :dim] = jnp.broadcast_to(t[:, None, :], (vocab, _PV, dim)).reshape(
            vocab * _PV, dim
        )
        o_ref[:, dim:] = jnp.broadcast_to(tpad[None], (vocab, _PV, dim)).reshape(
            vocab * _PV, dim
        )

    return pl.pallas_call(
        body,
        out_shape=jax.ShapeDtypeStruct((vocab * _PV, 2 * dim), table.dtype),
    )(table)


def kernel(poi_categories, table):
    batch, seq = poi_categories.shape
    vocab, dim = table.shape
    n = batch * seq
    np_ = n // 2

    idx = poi_categories.reshape(n).astype(jnp.int32)
    table2 = _build_pair_table(table)

    mesh = plsc.VectorSubcoreMesh(core_axis_name="c", subcore_axis_name="s")
    cp = pltpu.CompilerParams()
    if "needs_layout_passes" in pltpu.CompilerParams.__dataclass_fields__:
        cp = dataclasses.replace(cp, needs_layout_passes=False)

    nw = 32
    per_w = np_ // nw
    ic = 3200
    nsc = per_w // ic
    isteps = ic // _W

    @pl.kernel(
        out_type=jax.ShapeDtypeStruct((n, dim), table.dtype),
        mesh=mesh,
        scratch_types=[
            pltpu.VMEM((2 * 2 * ic,), jnp.int32),
            pltpu.VMEM((_W,), jnp.int32),
            pltpu.VMEM((_W,), jnp.int32),
            pltpu.VMEM((2 * _W, dim), jnp.float32),
            pltpu.VMEM((2 * _W, dim), jnp.float32),
            pltpu.SemaphoreType.DMA((2,)),
            pltpu.SemaphoreType.DMA((2,)),
            pltpu.SemaphoreType.DMA((2,)),
        ],
        compiler_params=cp,
    )
    def _gather(
        table_hbm, idx_hbm, out_hbm, idx_v, pid0_v, pid1_v, rows0_v, rows1_v,
        isem, gsem, wsem,
    ):
        pid_bufs = (pid0_v, pid1_v)
        rows_bufs = (rows0_v, rows1_v)
        wid = lax.axis_index("s") * 2 + lax.axis_index("c")
        base_i = wid * per_w * 2
        iota = lax.iota(jnp.int32, 16)

        def idx_load(c, jc):
            return pltpu.make_async_copy(
                idx_hbm.at[pl.ds(base_i + c * 2 * ic, 2 * ic)],
                idx_v.at[pl.ds(jc * 2 * ic, 2 * ic)],
                isem.at[jc],
            )

        def compute_pid(jc, s, slot):
            for g in range(_W // 16):
                pos = jc * 2 * ic + 2 * (s * _W + g * 16 + iota)
                a = plsc.load_gather(idx_v, [pos])
                b = plsc.load_gather(idx_v, [pos + 1])
                pid_bufs[slot][pl.ds(g * 16, 16)] = a * _PV + b

        def gather(slot):
            return pltpu.make_async_copy(
                table_hbm.at[pid_bufs[slot]],
                rows_bufs[slot].reshape(_W, 2 * dim),
                gsem.at[slot],
            )

        def writeback(c, s, slot):
            dst = out_hbm.at[pl.ds(base_i + c * 2 * ic + s * 2 * _W, 2 * _W)]
            return pltpu.make_async_copy(rows_bufs[slot], dst, wsem.at[slot])

        idx_load(0, 0).start()

        @pl.loop(0, nsc)
        def _(c):
            jc = lax.rem(c, 2)
            idx_load(c, jc).wait()

            @pl.when(c + 1 < nsc)
            def _():
                idx_load(c + 1, lax.rem(c + 1, 2)).start()

            @pl.when(c > 0)
            def _():
                writeback(c, 0, 0).wait()

            compute_pid(jc, 0, 0)
            gather(0).start()

            @pl.loop(0, isteps - 1, step=2)
            def _(r):
                for b in range(2):
                    s = r + b
                    gather(b).wait()
                    writeback(c, s, b).start()
                    tb = 1 - b

                    @pl.when((c > 0) | (s >= 1))
                    def _(tb=tb):
                        writeback(c, 0, tb).wait()

                    compute_pid(jc, s + 1, tb)
                    gather(tb).start()

            gather(0).wait()
            writeback(c, isteps - 1, 0).start()

        writeback(0, 0, 0).wait()
        writeback(0, 0, 1).wait()

    out = _gather(table2, idx)
    return out.reshape(batch, seq, dim)
